# baseline (speedup 1.0000x reference)
import math

import jax
import jax.numpy as jnp
from jax import lax
from jax.experimental import pallas as pl
from jax.experimental.pallas import tpu as pltpu
from jax.experimental.pallas import tpu_sc as plsc

_B = 2
_S = 512
_L_CMP = 32
_L_SLC = 32
_L_WIN = 128
_DIM = 512
_H = 8
_STRIDE = 16
_TOPK = 2
_HD = _DIM // _H
_KV_CMP = (_S - _L_CMP) // _STRIDE + 1
_KV_SLC = _S // _L_SLC
_SCALE = 1.0 / math.sqrt(_HD)
_NEG_INF = float("-inf")
_NSEL = _TOPK * _L_SLC


def _softmax(s):
    m = jnp.max(s, axis=-1, keepdims=True)
    e = jnp.exp(s - m)
    return e / jnp.sum(e, axis=-1, keepdims=True)


def _slc_map():
    rr = lax.broadcasted_iota(jnp.int32, (_KV_CMP, _KV_SLC), 0)
    jj = lax.broadcasted_iota(jnp.int32, (_KV_CMP, _KV_SLC), 1)
    return ((rr == 2 * jj).astype(jnp.float32)
            + 2.0 * (rr == 2 * jj + 1).astype(jnp.float32)
            + (rr == 2 * jj + 2).astype(jnp.float32))


def _stage_a(x_ref, wc_ref, bc_ref, wk_ref, bk_ref, wv_ref, bv_ref,
             q_ref, kvtab_ref, kcmp_ref, vcmp_ref, pslc_ref):
    x = x_ref[0]
    qkv = jnp.dot(x, wc_ref[...].T, preferred_element_type=jnp.float32)
    qkv = qkv + bc_ref[...]
    q = qkv[:, :_DIM]
    k = qkv[:, _DIM:2 * _DIM]
    v = qkv[:, 2 * _DIM:]
    q_ref[0] = q

    r = lax.broadcasted_iota(jnp.int32, (_KV_CMP, _S), 0)
    c = lax.broadcasted_iota(jnp.int32, (_KV_CMP, _S), 1)
    off = c - r * _STRIDE
    mk = jnp.zeros((_KV_CMP, _S), dtype=jnp.float32)
    mv = jnp.zeros((_KV_CMP, _S), dtype=jnp.float32)
    for l in range(_L_CMP):
        sel = (off == l).astype(jnp.float32)
        mk = mk + sel * wk_ref[0, l]
        mv = mv + sel * wv_ref[0, l]
    k_cmp = jnp.dot(mk, k, preferred_element_type=jnp.float32) + bk_ref[0, 0]
    v_cmp = jnp.dot(mv, v, preferred_element_type=jnp.float32) + bv_ref[0, 0]
    kcmp_ref[0] = k_cmp
    vcmp_ref[0] = v_cmp

    m_slc = _slc_map()
    q_last = q[_S - 1:_S, :]
    pslc_rows = []
    for h in range(_H):
        c0 = h * _HD
        kvtab_ref[pl.ds(h * _S, _S), :] = jnp.concatenate(
            [k[:, c0:c0 + _HD], v[:, c0:c0 + _HD]], axis=1)
        cs = jnp.dot(q_last[:, c0:c0 + _HD], k_cmp[:, c0:c0 + _HD].T,
                     preferred_element_type=jnp.float32) * _SCALE
        p = _softmax(cs)
        pslc_rows.append(jnp.dot(p, m_slc,
                                 preferred_element_type=jnp.float32))
    pslc_ref[0] = jnp.concatenate(pslc_rows, axis=0)


def _stage_b(x_ref, wg_ref, bg_ref, q_ref, kvtab_ref, kcmp_ref, vcmp_ref,
             out_ref):
    x = x_ref[0]
    q = q_ref[0]
    k_cmp = kcmp_ref[0]
    v_cmp = vcmp_ref[0]
    gate = jnp.dot(x, wg_ref[...].T, preferred_element_type=jnp.float32)
    gate = gate + bg_ref[...]
    g0 = gate[:, 0:1]
    g2 = gate[:, 2:3]

    ii_c = lax.broadcasted_iota(jnp.int32, (_S, _KV_CMP), 0)
    jj_c = lax.broadcasted_iota(jnp.int32, (_S, _KV_CMP), 1)
    cmp_valid = jj_c < ii_c

    n_t = _S // 128
    win_valid_t = []
    for t in range(n_t):
        slab0 = max(0, (t - 1) * 128)
        ii = lax.broadcasted_iota(jnp.int32, (128, 256), 0) + t * 128
        jj = lax.broadcasted_iota(jnp.int32, (128, 256), 1) + slab0
        win_valid_t.append((jj <= ii) & (jj >= ii - _L_WIN))

    for h in range(_H):
        c0 = h * _HD
        qh = q[:, c0:c0 + _HD]
        kh = kvtab_ref[pl.ds(h * _S, _S), :_HD]
        vh = kvtab_ref[pl.ds(h * _S, _S), _HD:]

        cs = jnp.dot(qh, k_cmp[:, c0:c0 + _HD].T,
                     preferred_element_type=jnp.float32) * _SCALE
        cs = jnp.where(cmp_valid, cs, _NEG_INF)
        p_cmp = _softmax(cs)
        cmp_o = jnp.dot(p_cmp, v_cmp[:, c0:c0 + _HD],
                        preferred_element_type=jnp.float32)

        for t in range(n_t):
            slab0 = max(0, (t - 1) * 128)
            qt = qh[t * 128:(t + 1) * 128, :]
            kt = kh[slab0:slab0 + 256, :]
            vt = vh[slab0:slab0 + 256, :]
            ws = jnp.dot(qt, kt.T, preferred_element_type=jnp.float32)
            ws = jnp.where(win_valid_t[t], ws, _NEG_INF) * _SCALE
            p_win = _softmax(ws)
            win_o = jnp.dot(p_win, vt, preferred_element_type=jnp.float32)
            r0 = t * 128
            out_ref[0, r0:r0 + 128, c0:c0 + _HD] = (
                g0[r0:r0 + 128] * cmp_o[r0:r0 + 128, :]
                + g2[r0:r0 + 128] * win_o)


def _bcast_max(x):
    c = plsc.cummax(x)
    return plsc.cummax(lax.rev(c, (0,)))


def _argmax_low(p, lane):
    m_all = _bcast_max(p)
    cand = jnp.where(p == m_all, -lane, -_KV_SLC)
    return -_bcast_max(cand)


def _sc_body(kvtab, pslc, kvsum_o, pslc_v, idx_v, slot_v, kvacc, shared):
    b = lax.axis_index("c")
    s = lax.axis_index("s")
    lane = lax.broadcasted_iota(jnp.int32, (16,), 0)

    @pl.when(s < _H)
    def _():
        h = s
        pltpu.sync_copy(pslc.at[b, h], pslc_v)
        p = pslc_v[...]
        i0 = _argmax_low(p, lane)
        pm = jnp.where(lane == i0, _NEG_INF, p)
        i1 = _argmax_low(pm, lane)
        base = (b * _H + h) * _S
        r0 = base + i0 * _L_SLC
        r1 = base + i1 * _L_SLC
        idx_v[pl.ds(0, 16)] = r0 + lane
        idx_v[pl.ds(16, 16)] = r0 + 16 + lane
        idx_v[pl.ds(32, 16)] = r1 + lane
        idx_v[pl.ds(48, 16)] = r1 + 16 + lane
        slot_v[pl.ds(0, 16)] = lane
        slot_v[pl.ds(16, 16)] = 16 + lane
        slot_v[pl.ds(32, 16)] = 32 + lane
        slot_v[pl.ds(48, 16)] = 48 + lane
        pltpu.sync_copy(kvtab.at[idx_v], kvacc)

    @pl.when(s == 0)
    def _():
        pltpu.sync_copy(kvacc, shared)

    plsc.subcore_barrier()

    @pl.when((s >= 1) & (s < _H))
    def _():
        pltpu.sync_copy(kvacc, shared.at[slot_v], add=True)

    plsc.subcore_barrier()

    @pl.when(s == 0)
    def _():
        pltpu.sync_copy(shared, kvsum_o.at[b])


def _sc_select(kvtab2, pslc):
    fn = pl.kernel(
        _sc_body,
        out_type=[
            jax.ShapeDtypeStruct((_B, _NSEL, 2 * _HD), jnp.float32),
        ],
        mesh=plsc.VectorSubcoreMesh(core_axis_name="c", subcore_axis_name="s"),
        compiler_params=pltpu.CompilerParams(needs_layout_passes=False),
        scratch_types=[
            pltpu.VMEM((16,), jnp.float32),
            pltpu.VMEM((_NSEL,), jnp.int32),
            pltpu.VMEM((_NSEL,), jnp.int32),
            pltpu.VMEM((_NSEL, 2 * _HD), jnp.float32),
            pltpu.VMEM_SHARED((_NSEL, 2 * _HD), jnp.float32),
        ],
    )
    return fn(kvtab2, pslc)


def _stage_c(oin_ref, q_ref, xg_ref, wg_ref, bg_ref, kvsum_ref, out_ref):
    out_ref[...] = oin_ref[...]
    g1 = jnp.sum(xg_ref[0, 7:8, :] * wg_ref[1:2, :]) + bg_ref[0, 1]
    q_last = q_ref[0, 7:8, :]
    ks = kvsum_ref[0, :, :_HD]
    vs = kvsum_ref[0, :, _HD:]
    for h in range(_H):
        c0 = h * _HD
        a = jnp.dot(q_last[:, c0:c0 + _HD], ks.T,
                    preferred_element_type=jnp.float32) * _SCALE
        p = _softmax(a)
        o = jnp.dot(p, vs, preferred_element_type=jnp.float32)
        cur = out_ref[0, 7:8, c0:c0 + _HD]
        out_ref[0, 7:8, c0:c0 + _HD] = cur + g1 * o


def kernel(x, Wc, bc, Wk, bk, Wv, bv, Wg, bg):
    bc2 = bc.reshape(1, 3 * _DIM)
    bk2 = bk.reshape(1, 1)
    bv2 = bv.reshape(1, 1)
    bg2 = bg.reshape(1, 3)

    def full(shape):
        return pl.BlockSpec(shape, lambda b: (0,) * len(shape))

    q, kvtab, kcmp, vcmp, pslc = pl.pallas_call(
        _stage_a,
        grid=(_B,),
        in_specs=[
            pl.BlockSpec((1, _S, _DIM), lambda b: (b, 0, 0)),
            full((3 * _DIM, _DIM)),
            full((1, 3 * _DIM)),
            full((1, _L_CMP)),
            full((1, 1)),
            full((1, _L_CMP)),
            full((1, 1)),
        ],
        out_specs=[
            pl.BlockSpec((1, _S, _DIM), lambda b: (b, 0, 0)),
            pl.BlockSpec((_H * _S, 2 * _HD), lambda b: (b, 0)),
            pl.BlockSpec((1, _KV_CMP, _DIM), lambda b: (b, 0, 0)),
            pl.BlockSpec((1, _KV_CMP, _DIM), lambda b: (b, 0, 0)),
            pl.BlockSpec((1, _H, _KV_SLC), lambda b: (b, 0, 0)),
        ],
        out_shape=[
            jax.ShapeDtypeStruct((_B, _S, _DIM), jnp.float32),
            jax.ShapeDtypeStruct((_B * _H * _S, 2 * _HD), jnp.float32),
            jax.ShapeDtypeStruct((_B, _KV_CMP, _DIM), jnp.float32),
            jax.ShapeDtypeStruct((_B, _KV_CMP, _DIM), jnp.float32),
            jax.ShapeDtypeStruct((_B, _H, _KV_SLC), jnp.float32),
        ],
        compiler_params=pltpu.CompilerParams(
            dimension_semantics=("parallel",),
        ),
    )(x, Wc, bc2, Wk, bk2, Wv, bv2)

    (kvsum,) = _sc_select(kvtab, pslc)

    out = pl.pallas_call(
        _stage_b,
        grid=(_B,),
        in_specs=[
            pl.BlockSpec((1, _S, _DIM), lambda b: (b, 0, 0)),
            full((3, _DIM)),
            full((1, 3)),
            pl.BlockSpec((1, _S, _DIM), lambda b: (b, 0, 0)),
            pl.BlockSpec((_H * _S, 2 * _HD), lambda b: (b, 0)),
            pl.BlockSpec((1, _KV_CMP, _DIM), lambda b: (b, 0, 0)),
            pl.BlockSpec((1, _KV_CMP, _DIM), lambda b: (b, 0, 0)),
        ],
        out_specs=pl.BlockSpec((1, _S, _DIM), lambda b: (b, 0, 0)),
        out_shape=jax.ShapeDtypeStruct((_B, _S, _DIM), jnp.float32),
        compiler_params=pltpu.CompilerParams(
            dimension_semantics=("parallel",),
        ),
    )(x, Wg, bg2, q, kvtab, kcmp, vcmp)

    row_blk = pl.BlockSpec((1, 8, _DIM), lambda b: (b, 1, 0))
    out = pl.pallas_call(
        _stage_c,
        grid=(_B,),
        in_specs=[
            row_blk,
            pl.BlockSpec((1, 8, _DIM), lambda b: (b, 63, 0)),
            pl.BlockSpec((1, 8, _DIM), lambda b: (b, 1, 0)),
            full((3, _DIM)),
            full((1, 3)),
            pl.BlockSpec((1, _NSEL, 2 * _HD), lambda b: (b, 0, 0)),
        ],
        out_specs=row_blk,
        out_shape=jax.ShapeDtypeStruct((_B, _S, _DIM), jnp.float32),
        input_output_aliases={0: 0},
        compiler_params=pltpu.CompilerParams(
            dimension_semantics=("parallel",),
        ),
    )(out, q, x, Wg, bg2, kvsum)
    return out


# speedup vs baseline: 1.2285x; 1.2285x over previous
import math

import jax
import jax.numpy as jnp
from jax import lax
from jax.experimental import pallas as pl
from jax.experimental.pallas import tpu as pltpu
from jax.experimental.pallas import tpu_sc as plsc

_B = 2
_S = 512
_L_CMP = 32
_L_SLC = 32
_L_WIN = 128
_DIM = 512
_H = 8
_STRIDE = 16
_TOPK = 2
_HD = _DIM // _H
_KV_CMP = (_S - _L_CMP) // _STRIDE + 1
_KV_SLC = _S // _L_SLC
_SCALE = 1.0 / math.sqrt(_HD)
_NEG_INF = float("-inf")
_NSEL = _TOPK * _L_SLC


def _softmax(s):
    m = jnp.max(s, axis=-1, keepdims=True)
    e = jnp.exp(s - m)
    return e / jnp.sum(e, axis=-1, keepdims=True)


def _slc_map():
    rr = lax.broadcasted_iota(jnp.int32, (_KV_CMP, _KV_SLC), 0)
    jj = lax.broadcasted_iota(jnp.int32, (_KV_CMP, _KV_SLC), 1)
    return ((rr == 2 * jj).astype(jnp.float32)
            + 2.0 * (rr == 2 * jj + 1).astype(jnp.float32)
            + (rr == 2 * jj + 2).astype(jnp.float32))


def _stage_ab(x_ref, wc_ref, bc_ref, wk_ref, bk_ref, wv_ref, bv_ref,
              wg_ref, bg_ref, out_ref, kvtab_ref, pslc_ref, qrow_ref):
    x = x_ref[0]
    qkv = jnp.dot(x, wc_ref[...].T, preferred_element_type=jnp.float32)
    qkv = qkv + bc_ref[...]
    q = qkv[:, :_DIM]
    k = qkv[:, _DIM:2 * _DIM]
    v = qkv[:, 2 * _DIM:]
    gate = jnp.dot(x, wg_ref[...].T, preferred_element_type=jnp.float32)
    gate = gate + bg_ref[...]
    g0 = gate[:, 0:1]
    g2 = gate[:, 2:3]

    r = lax.broadcasted_iota(jnp.int32, (_KV_CMP, _S), 0)
    c = lax.broadcasted_iota(jnp.int32, (_KV_CMP, _S), 1)
    off = c - r * _STRIDE
    mk = jnp.zeros((_KV_CMP, _S), dtype=jnp.float32)
    mv = jnp.zeros((_KV_CMP, _S), dtype=jnp.float32)
    for l in range(_L_CMP):
        sel = (off == l).astype(jnp.float32)
        mk = mk + sel * wk_ref[0, l]
        mv = mv + sel * wv_ref[0, l]
    k_cmp = jnp.dot(mk, k, preferred_element_type=jnp.float32) + bk_ref[0, 0]
    v_cmp = jnp.dot(mv, v, preferred_element_type=jnp.float32) + bv_ref[0, 0]

    ii_c = lax.broadcasted_iota(jnp.int32, (_S, _KV_CMP), 0)
    jj_c = lax.broadcasted_iota(jnp.int32, (_S, _KV_CMP), 1)
    cmp_valid = jj_c < ii_c
    m_slc = _slc_map()

    n_t = _S // 128
    win_valid_t = []
    for t in range(n_t):
        slab0 = max(0, (t - 1) * 128)
        ii = lax.broadcasted_iota(jnp.int32, (128, 256), 0) + t * 128
        jj = lax.broadcasted_iota(jnp.int32, (128, 256), 1) + slab0
        win_valid_t.append((jj <= ii) & (jj >= ii - _L_WIN))

    pslc_rows = []
    for h in range(_H):
        c0 = h * _HD
        qh = q[:, c0:c0 + _HD]
        kh = k[:, c0:c0 + _HD]
        vh = v[:, c0:c0 + _HD]
        kvtab_ref[pl.ds(h * _S, _S), :] = jnp.concatenate([kh, vh], axis=1)

        cs = jnp.dot(qh, k_cmp[:, c0:c0 + _HD].T,
                     preferred_element_type=jnp.float32) * _SCALE
        cs = jnp.where(cmp_valid, cs, _NEG_INF)
        p_cmp = _softmax(cs)
        cmp_o = jnp.dot(p_cmp, v_cmp[:, c0:c0 + _HD],
                        preferred_element_type=jnp.float32)

        for t in range(n_t):
            slab0 = max(0, (t - 1) * 128)
            qt = qh[t * 128:(t + 1) * 128, :]
            kt = kh[slab0:slab0 + 256, :]
            vt = vh[slab0:slab0 + 256, :]
            ws = jnp.dot(qt, kt.T, preferred_element_type=jnp.float32)
            ws = jnp.where(win_valid_t[t], ws, _NEG_INF) * _SCALE
            p_win = _softmax(ws)
            win_o = jnp.dot(p_win, vt, preferred_element_type=jnp.float32)
            r0 = t * 128
            out_ref[0, r0:r0 + 128, c0:c0 + _HD] = (
                g0[r0:r0 + 128] * cmp_o[r0:r0 + 128, :]
                + g2[r0:r0 + 128] * win_o)

        pslc_rows.append(jnp.dot(p_cmp[_S - 1:_S, :], m_slc,
                                 preferred_element_type=jnp.float32))
    pslc_ref[0] = jnp.concatenate(pslc_rows, axis=0)
    qrow_ref[0] = jnp.broadcast_to(q[_S - 1:_S, :], (8, _DIM))


def _bcast_max(x):
    c = plsc.cummax(x)
    return plsc.cummax(lax.rev(c, (0,)))


def _argmax_low(p, lane):
    m_all = _bcast_max(p)
    cand = jnp.where(p == m_all, -lane, -_KV_SLC)
    return -_bcast_max(cand)


def _sc_body(kvtab, pslc, kvsum_o, pslc_v, idx_v, slot_v, kvacc, shared):
    b = lax.axis_index("c")
    s = lax.axis_index("s")
    lane = lax.broadcasted_iota(jnp.int32, (16,), 0)

    @pl.when(s < _H)
    def _():
        h = s
        pltpu.sync_copy(pslc.at[b, h], pslc_v)
        p = pslc_v[...]
        i0 = _argmax_low(p, lane)
        pm = jnp.where(lane == i0, _NEG_INF, p)
        i1 = _argmax_low(pm, lane)
        base = (b * _H + h) * _S
        r0 = base + i0 * _L_SLC
        r1 = base + i1 * _L_SLC
        idx_v[pl.ds(0, 16)] = r0 + lane
        idx_v[pl.ds(16, 16)] = r0 + 16 + lane
        idx_v[pl.ds(32, 16)] = r1 + lane
        idx_v[pl.ds(48, 16)] = r1 + 16 + lane
        slot_v[pl.ds(0, 16)] = lane
        slot_v[pl.ds(16, 16)] = 16 + lane
        slot_v[pl.ds(32, 16)] = 32 + lane
        slot_v[pl.ds(48, 16)] = 48 + lane
        pltpu.sync_copy(kvtab.at[idx_v], kvacc)

    @pl.when(s == 0)
    def _():
        pltpu.sync_copy(kvacc, shared)

    plsc.subcore_barrier()

    @pl.when((s >= 1) & (s < _H))
    def _():
        pltpu.sync_copy(kvacc, shared.at[slot_v], add=True)

    plsc.subcore_barrier()

    @pl.when(s == 0)
    def _():
        pltpu.sync_copy(shared, kvsum_o.at[b])


def _sc_select(kvtab2, pslc):
    fn = pl.kernel(
        _sc_body,
        out_type=[
            jax.ShapeDtypeStruct((_B, _NSEL, 2 * _HD), jnp.float32),
        ],
        mesh=plsc.VectorSubcoreMesh(core_axis_name="c", subcore_axis_name="s"),
        compiler_params=pltpu.CompilerParams(needs_layout_passes=False),
        scratch_types=[
            pltpu.VMEM((16,), jnp.float32),
            pltpu.VMEM((_NSEL,), jnp.int32),
            pltpu.VMEM((_NSEL,), jnp.int32),
            pltpu.VMEM((_NSEL, 2 * _HD), jnp.float32),
            pltpu.VMEM_SHARED((_NSEL, 2 * _HD), jnp.float32),
        ],
    )
    return fn(kvtab2, pslc)


def _stage_c(oin_ref, qrow_ref, xg_ref, wg_ref, bg_ref, kvsum_ref, out_ref):
    out_ref[...] = oin_ref[...]
    for b in range(_B):
        g1 = jnp.sum(xg_ref[b, 7:8, :] * wg_ref[1:2, :]) + bg_ref[0, 1]
        q_last = qrow_ref[b, 0:1, :]
        ks = kvsum_ref[b, :, :_HD]
        vs = kvsum_ref[b, :, _HD:]
        for h in range(_H):
            c0 = h * _HD
            a = jnp.dot(q_last[:, c0:c0 + _HD], ks.T,
                        preferred_element_type=jnp.float32) * _SCALE
            p = _softmax(a)
            o = jnp.dot(p, vs, preferred_element_type=jnp.float32)
            cur = out_ref[b, 7:8, c0:c0 + _HD]
            out_ref[b, 7:8, c0:c0 + _HD] = cur + g1 * o


def kernel(x, Wc, bc, Wk, bk, Wv, bv, Wg, bg):
    bc2 = bc.reshape(1, 3 * _DIM)
    bk2 = bk.reshape(1, 1)
    bv2 = bv.reshape(1, 1)
    bg2 = bg.reshape(1, 3)

    def full(shape):
        return pl.BlockSpec(shape, lambda b: (0,) * len(shape))

    out, kvtab, pslc, qrow = pl.pallas_call(
        _stage_ab,
        grid=(_B,),
        in_specs=[
            pl.BlockSpec((1, _S, _DIM), lambda b: (b, 0, 0)),
            full((3 * _DIM, _DIM)),
            full((1, 3 * _DIM)),
            full((1, _L_CMP)),
            full((1, 1)),
            full((1, _L_CMP)),
            full((1, 1)),
            full((3, _DIM)),
            full((1, 3)),
        ],
        out_specs=[
            pl.BlockSpec((1, _S, _DIM), lambda b: (b, 0, 0)),
            pl.BlockSpec((_H * _S, 2 * _HD), lambda b: (b, 0)),
            pl.BlockSpec((1, _H, _KV_SLC), lambda b: (b, 0, 0)),
            pl.BlockSpec((1, 8, _DIM), lambda b: (b, 0, 0)),
        ],
        out_shape=[
            jax.ShapeDtypeStruct((_B, _S, _DIM), jnp.float32),
            jax.ShapeDtypeStruct((_B * _H * _S, 2 * _HD), jnp.float32),
            jax.ShapeDtypeStruct((_B, _H, _KV_SLC), jnp.float32),
            jax.ShapeDtypeStruct((_B, 8, _DIM), jnp.float32),
        ],
        compiler_params=pltpu.CompilerParams(
            dimension_semantics=("parallel",),
        ),
    )(x, Wc, bc2, Wk, bk2, Wv, bv2, Wg, bg2)

    (kvsum,) = _sc_select(kvtab, pslc)

    row_blk = pl.BlockSpec((_B, 8, _DIM), lambda i: (0, 1, 0))
    out = pl.pallas_call(
        _stage_c,
        grid=(1,),
        in_specs=[
            row_blk,
            pl.BlockSpec((_B, 8, _DIM), lambda i: (0, 0, 0)),
            pl.BlockSpec((_B, 8, _DIM), lambda i: (0, 1, 0)),
            pl.BlockSpec((3, _DIM), lambda i: (0, 0)),
            pl.BlockSpec((1, 3), lambda i: (0, 0)),
            pl.BlockSpec((_B, _NSEL, 2 * _HD), lambda i: (0, 0, 0)),
        ],
        out_specs=row_blk,
        out_shape=jax.ShapeDtypeStruct((_B, _S, _DIM), jnp.float32),
        input_output_aliases={0: 0},
    )(out, qrow, x, Wg, bg2, kvsum)
    return out
